# Initial kernel scaffold; baseline (speedup 1.0000x reference)
import functools

import jax
import jax.numpy as jnp
from jax import lax
from jax.experimental import pallas as pl
from jax.experimental.pallas import tpu as pltpu
from jax.experimental.pallas import tpu_sc as plsc

BATCH = 16384
N_FIELDS = 26
EMBED_DIM = 64

NUM_CORES = 2
NUM_SUBCORES = 16
NUM_WORKERS = NUM_CORES * NUM_SUBCORES
ROWS_PER_WORKER = BATCH // NUM_WORKERS
ROWS_PER_CHUNK = 4
IDX_PER_CHUNK = ROWS_PER_CHUNK * N_FIELDS
CHUNKS = ROWS_PER_WORKER // ROWS_PER_CHUNK
NBUF = 2
LANES = 16
COL_GROUPS = EMBED_DIM // LANES


def _sc_classifier(x3, table):
    mesh = plsc.VectorSubcoreMesh(core_axis_name="c", subcore_axis_name="s")

    @functools.partial(
        pl.kernel,
        mesh=mesh,
        out_type=jax.ShapeDtypeStruct((BATCH, EMBED_DIM), jnp.float32),
        scratch_types=(
            [pltpu.VMEM((CHUNKS, IDX_PER_CHUNK), jnp.int32)]
            + [pltpu.VMEM((IDX_PER_CHUNK, EMBED_DIM), jnp.float32)
               for _ in range(NBUF)]
            + [pltpu.VMEM((ROWS_PER_WORKER, EMBED_DIM), jnp.float32)]
            + [pltpu.SemaphoreType.DMA for _ in range(NBUF)]
        ),
    )
    def k(x_hbm, table_hbm, out_hbm, idx_v, gbuf0, gbuf1, out_v, sem0, sem1):
        gbufs = (gbuf0, gbuf1)
        sems = (sem0, sem1)
        wid = lax.axis_index("s") * NUM_CORES + lax.axis_index("c")

        pltpu.sync_copy(x_hbm.at[wid], idx_v)

        for b in range(NBUF):
            pltpu.async_copy(table_hbm.at[idx_v.at[b]], gbufs[b], sems[b])

        def chunk_body(t, carry):
            for b in range(NBUF):
                c = t * NBUF + b
                pltpu.make_async_copy(
                    table_hbm.at[idx_v.at[c]], gbufs[b], sems[b]
                ).wait()
                gb = gbufs[b]
                for r in range(ROWS_PER_CHUNK):
                    acc = [gb[r * N_FIELDS, pl.ds(g * LANES, LANES)]
                           for g in range(COL_GROUPS)]
                    for f in range(1, N_FIELDS):
                        for g in range(COL_GROUPS):
                            acc[g] = acc[g] + gb[
                                r * N_FIELDS + f, pl.ds(g * LANES, LANES)]
                    m = jnp.max(jnp.maximum(jnp.maximum(acc[0], acc[1]),
                                            jnp.maximum(acc[2], acc[3])))
                    ex = [jnp.exp(a - m) for a in acc]
                    s = jnp.sum(ex[0] + ex[1] + ex[2] + ex[3])
                    inv = 1.0 / s
                    row = c * ROWS_PER_CHUNK + r
                    for g in range(COL_GROUPS):
                        out_v[row, pl.ds(g * LANES, LANES)] = ex[g] * inv

                nxt = c + NBUF

                @pl.when(nxt < CHUNKS)
                def _():
                    pltpu.async_copy(
                        table_hbm.at[idx_v.at[nxt]], gbufs[b], sems[b])

            return carry

        lax.fori_loop(0, CHUNKS // NBUF, chunk_body, 0)

        pltpu.sync_copy(
            out_v, out_hbm.at[pl.ds(wid * ROWS_PER_WORKER, ROWS_PER_WORKER)])

    return k(x3, table)


def kernel(x, table):
    x3 = x.astype(jnp.int32).reshape(NUM_WORKERS, CHUNKS, IDX_PER_CHUNK)
    return _sc_classifier(x3, table)


# speedup vs baseline: 1.3487x; 1.3487x over previous
import functools

import jax
import jax.numpy as jnp
from jax import lax
from jax.experimental import pallas as pl
from jax.experimental.pallas import tpu as pltpu
from jax.experimental.pallas import tpu_sc as plsc

BATCH = 16384
N_FIELDS = 26
EMBED_DIM = 64

NUM_CORES = 2
NUM_SUBCORES = 16
NUM_WORKERS = NUM_CORES * NUM_SUBCORES
ROWS_PER_WORKER = BATCH // NUM_WORKERS
ROWS_PER_CHUNK = 4
IDX_PER_CHUNK = ROWS_PER_CHUNK * N_FIELDS
CHUNKS = ROWS_PER_WORKER // ROWS_PER_CHUNK
NBUF = 2
LANES = 16
COL_GROUPS = EMBED_DIM // LANES
ROW_GROUPS = ROWS_PER_WORKER // LANES


def _sc_classifier(x3, table):
    mesh = plsc.VectorSubcoreMesh(core_axis_name="c", subcore_axis_name="s")

    @functools.partial(
        pl.kernel,
        mesh=mesh,
        compiler_params=pltpu.CompilerParams(
            use_tc_tiling_on_sc=False, needs_layout_passes=False),
        out_type=jax.ShapeDtypeStruct((BATCH, EMBED_DIM), jnp.float32),
        scratch_types=(
            [pltpu.VMEM((CHUNKS, IDX_PER_CHUNK), jnp.int32)]
            + [pltpu.VMEM((IDX_PER_CHUNK, EMBED_DIM), jnp.float32)
               for _ in range(NBUF)]
            + [pltpu.VMEM((EMBED_DIM, ROWS_PER_WORKER), jnp.float32)]
            + [pltpu.VMEM((ROWS_PER_WORKER, EMBED_DIM), jnp.float32)]
            + [pltpu.SemaphoreType.DMA for _ in range(NBUF)]
        ),
    )
    def k(x_hbm, table_hbm, out_hbm, idx_v, gbuf0, gbuf1, acc_t, out_v,
          sem0, sem1):
        gbufs = (gbuf0, gbuf1)
        sems = (sem0, sem1)
        wid = lax.axis_index("s") * NUM_CORES + lax.axis_index("c")
        lane = lax.iota(jnp.int32, LANES)

        pltpu.sync_copy(x_hbm.at[wid], idx_v)

        for b in range(NBUF):
            pltpu.async_copy(table_hbm.at[idx_v.at[b]], gbufs[b], sems[b])

        def chunk_body(t, carry):
            for b in range(NBUF):
                c = t * NBUF + b
                pltpu.make_async_copy(
                    table_hbm.at[idx_v.at[c]], gbufs[b], sems[b]
                ).wait()
                gb = gbufs[b]
                for r in range(ROWS_PER_CHUNK):
                    acc = [gb[r * N_FIELDS, pl.ds(g * LANES, LANES)]
                           for g in range(COL_GROUPS)]
                    for f in range(1, N_FIELDS):
                        for g in range(COL_GROUPS):
                            acc[g] = acc[g] + gb[
                                r * N_FIELDS + f, pl.ds(g * LANES, LANES)]
                    row = jnp.full((LANES,), c * ROWS_PER_CHUNK + r,
                                   dtype=jnp.int32)
                    for g in range(COL_GROUPS):
                        plsc.store_scatter(
                            acc_t, [lane + (g * LANES), row], acc[g])

                nxt = c + NBUF

                @pl.when(nxt < CHUNKS)
                def _():
                    pltpu.async_copy(
                        table_hbm.at[idx_v.at[nxt]], gbufs[b], sems[b])

            return carry

        lax.fori_loop(0, CHUNKS // NBUF, chunk_body, 0)

        def softmax_body(g, carry):
            cols = [acc_t[cc, pl.ds(g * LANES, LANES)]
                    for cc in range(EMBED_DIM)]
            m = cols[0]
            for cc in range(1, EMBED_DIM):
                m = jnp.maximum(m, cols[cc])
            ex = [jnp.exp(v - m) for v in cols]
            s = ex[0]
            for cc in range(1, EMBED_DIM):
                s = s + ex[cc]
            inv = 1.0 / s
            rows = g * LANES + lane
            for cc in range(EMBED_DIM):
                plsc.store_scatter(
                    out_v, [rows, jnp.full((LANES,), cc, dtype=jnp.int32)],
                    ex[cc] * inv)
            return carry

        lax.fori_loop(0, ROW_GROUPS, softmax_body, 0)

        pltpu.sync_copy(
            out_v, out_hbm.at[pl.ds(wid * ROWS_PER_WORKER, ROWS_PER_WORKER)])

    return k(x3, table)


def kernel(x, table):
    x3 = x.astype(jnp.int32).reshape(NUM_WORKERS, CHUNKS, IDX_PER_CHUNK)
    return _sc_classifier(x3, table)
